# Initial kernel scaffold; baseline (speedup 1.0000x reference)
import jax
import jax.numpy as jnp
from jax.experimental import pallas as pl


def kernel(input_ids, attn_mask, wte):
    raise NotImplementedError("write your pallas kernel here")


# speedup vs baseline: 1.7902x; 1.7902x over previous
import functools

import jax
import jax.numpy as jnp
from jax import lax
from jax.experimental import pallas as pl
from jax.experimental.pallas import tpu as pltpu
from jax.experimental.pallas import tpu_sc as plsc

VOCAB = 50257
EMBED = 768
B = 128
L = 512

NC = 2
NS = 16
NW = NC * NS
N = B * L
PER_W = N // NW
C = 64
NCH = PER_W // C

_mesh = plsc.VectorSubcoreMesh(
    core_axis_name="c", subcore_axis_name="s", num_cores=NC, num_subcores=NS
)


@functools.partial(
    pl.kernel,
    out_type=jax.ShapeDtypeStruct((NW, NCH, C, EMBED), jnp.float32),
    mesh=_mesh,
    scratch_types=[
        pltpu.VMEM((NCH, C), jnp.int32),
        pltpu.VMEM((2, C, EMBED), jnp.float32),
        pltpu.SemaphoreType.DMA,
        pltpu.SemaphoreType.DMA,
    ],
)
def _gather_kernel(ids_hbm, wte_hbm, out_hbm, idx_v, rows_v, sem0, sem1):
    wid = lax.axis_index("s") * NC + lax.axis_index("c")
    pltpu.sync_copy(ids_hbm.at[wid], idx_v)
    sems = (sem0, sem1)

    pltpu.async_copy(wte_hbm.at[idx_v.at[0]], rows_v.at[0], sem0)
    pltpu.async_copy(wte_hbm.at[idx_v.at[1]], rows_v.at[1], sem1)

    @pl.loop(0, NCH - 2, step=2)
    def _steady(cc):
        for b in range(2):
            c = cc + b
            pltpu.make_async_copy(
                wte_hbm.at[pl.ds(0, C)], rows_v.at[b], sems[b]
            ).wait()
            pltpu.sync_copy(rows_v.at[b], out_hbm.at[wid, c])
            pltpu.async_copy(wte_hbm.at[idx_v.at[c + 2]], rows_v.at[b], sems[b])

    for b in range(2):
        pltpu.make_async_copy(
            wte_hbm.at[pl.ds(0, C)], rows_v.at[b], sems[b]
        ).wait()
        pltpu.sync_copy(rows_v.at[b], out_hbm.at[wid, NCH - 2 + b])


def kernel(input_ids, attn_mask, wte):
    ids = input_ids.reshape(NW, NCH, C).astype(jnp.int32)
    out = _gather_kernel(ids, wte)
    return (out.reshape(B, L, EMBED), attn_mask)
